# Initial kernel scaffold; baseline (speedup 1.0000x reference)
import functools

import jax
import jax.numpy as jnp
from jax import lax
from jax.experimental import pallas as pl
from jax.experimental.pallas import tpu as pltpu
from jax.experimental.pallas import tpu_sc as plsc

_E = 16
_K = 2
_D = 1024
_FF = 768
_N = 4096
_BLK = 256
_NB = 48
_NSLOT = _NB * _BLK
_NBPAD = 64

_NW = 32
_TPW = _N // _NW
_CH1 = 64
_CH2 = 32


def _router_body(x_ref, w_ref, d1_ref, d2_ref, r1_ref, r2_ref, meta_ref):
    x = x_ref[...]
    w = w_ref[...]
    logits = lax.dot_general(x, w, (((1,), (1,)), ((), ())),
                             preferred_element_type=jnp.float32)
    lane = lax.broadcasted_iota(jnp.int32, (_N, _E), 1)
    m1 = jnp.max(logits, axis=1, keepdims=True)
    a1 = jnp.min(jnp.where(logits == m1, lane, _E), axis=1, keepdims=True)
    masked = jnp.where(lane == a1, jnp.finfo(jnp.float32).min, logits)
    m2 = jnp.max(masked, axis=1, keepdims=True)
    a2 = jnp.min(jnp.where(masked == m2, lane, _E), axis=1, keepdims=True)
    e2 = jnp.exp(m2 - m1)
    w1 = 1.0 / (1.0 + e2)
    w2 = e2 / (1.0 + e2)

    oh1 = lane == a1
    oh2 = lane == a2
    cnt = oh1.astype(jnp.int32) + oh2.astype(jnp.int32)
    cs = cnt
    k = 1
    while k < _N:
        cs = cs + jnp.concatenate(
            [jnp.zeros((k, _E), jnp.int32), cs[:-k, :]], axis=0)
        k *= 2
    excl = cs - cnt
    count = cs[_N - 1:_N, :]
    nblk = (count + (_BLK - 1)) // _BLK
    bs = nblk
    k = 1
    while k < _E:
        bs = bs + jnp.concatenate(
            [jnp.zeros((1, k), jnp.int32), bs[:, :-k]], axis=1)
        k *= 2
    bstart = bs - nblk
    pstart = bstart * _BLK

    d1 = jnp.sum(jnp.where(oh1, pstart + excl, 0), axis=1)
    d2 = jnp.sum(jnp.where(oh2, pstart + excl, 0), axis=1)
    d1_ref[...] = d1.reshape(_N // 128, 128)
    d2_ref[...] = d2.reshape(_N // 128, 128)
    r1_ref[...] = jnp.broadcast_to(w1, (_N, 1)).reshape(_N // 128, 128)
    r2_ref[...] = jnp.broadcast_to(w2, (_N, 1)).reshape(_N // 128, 128)

    bi = lax.broadcasted_iota(jnp.int32, (_NBPAD, _E), 0)
    lane_b = lax.broadcasted_iota(jnp.int32, (_NBPAD, _E), 1)
    act_mask = (bi >= bstart) & (bi < bstart + nblk)
    be = jnp.sum(jnp.where(act_mask, lane_b, 0), axis=1)
    active = jnp.sum(act_mask.astype(jnp.int32), axis=1)
    lane_r = lax.broadcasted_iota(jnp.int32, (1, _E), 1)
    lastexp = jnp.max(jnp.where(nblk > 0, lane_r, 0))
    be = jnp.where(active > 0, be, lastexp)
    meta_ref[...] = jnp.concatenate(
        [be.reshape(1, _NBPAD), active.reshape(1, _NBPAD)], axis=0)


def _router(x, router_w):
    return pl.pallas_call(
        _router_body,
        out_shape=[
            jax.ShapeDtypeStruct((_N // 128, 128), jnp.int32),
            jax.ShapeDtypeStruct((_N // 128, 128), jnp.int32),
            jax.ShapeDtypeStruct((_N // 128, 128), jnp.float32),
            jax.ShapeDtypeStruct((_N // 128, 128), jnp.float32),
            jax.ShapeDtypeStruct((2, _NBPAD), jnp.int32),
        ],
    )(x, router_w)


def _dispatch_body(x_hbm, d1_hbm, d2_hbm, out_hbm, idx_v, xbuf, sem):
    wid = lax.axis_index("s") * 2 + lax.axis_index("c")
    base = wid * _TPW
    for c in range(_TPW // _CH1):
        off = base + c * _CH1
        pltpu.sync_copy(d1_hbm.at[pl.ds(off, _CH1)], idx_v.at[0])
        pltpu.sync_copy(d2_hbm.at[pl.ds(off, _CH1)], idx_v.at[1])
        pltpu.sync_copy(x_hbm.at[pl.ds(off, _CH1)], xbuf)
        cp1 = pltpu.async_copy(xbuf, out_hbm.at[idx_v.at[0]], sem)
        cp2 = pltpu.async_copy(xbuf, out_hbm.at[idx_v.at[1]], sem)
        cp1.wait()
        cp2.wait()


_dispatch = functools.partial(
    pl.kernel,
    out_type=jax.ShapeDtypeStruct((_NSLOT, _D), jnp.float32),
    mesh=plsc.VectorSubcoreMesh(core_axis_name="c", subcore_axis_name="s"),
    scratch_types=[
        pltpu.VMEM((2, _CH1), jnp.int32),
        pltpu.VMEM((_CH1, _D), jnp.float32),
        pltpu.SemaphoreType.DMA,
    ],
)(_dispatch_body)


def _ffn_body(be_ref, act_ref, xs_ref, wg_ref, wu_ref, wd_ref, y_ref):
    i = pl.program_id(0)

    @pl.when(act_ref[i] > 0)
    def _():
        xb = xs_ref[...]
        g = lax.dot_general(xb, wg_ref[0], (((1,), (1,)), ((), ())),
                            preferred_element_type=jnp.float32)
        u = lax.dot_general(xb, wu_ref[0], (((1,), (1,)), ((), ())),
                            preferred_element_type=jnp.float32)
        h = (g * jax.nn.sigmoid(g)) * u
        y_ref[...] = lax.dot_general(h, wd_ref[0], (((1,), (1,)), ((), ())),
                                     preferred_element_type=jnp.float32)


def _ffn(be, act, xs, w_gate, w_up, w_down):
    grid_spec = pltpu.PrefetchScalarGridSpec(
        num_scalar_prefetch=2,
        grid=(_NB,),
        in_specs=[
            pl.BlockSpec((_BLK, _D), lambda i, be, act: (i, 0)),
            pl.BlockSpec((1, _FF, _D), lambda i, be, act: (be[i], 0, 0)),
            pl.BlockSpec((1, _FF, _D), lambda i, be, act: (be[i], 0, 0)),
            pl.BlockSpec((1, _D, _FF), lambda i, be, act: (be[i], 0, 0)),
        ],
        out_specs=pl.BlockSpec((_BLK, _D), lambda i, be, act: (i, 0)),
    )
    return pl.pallas_call(
        _ffn_body,
        grid_spec=grid_spec,
        out_shape=jax.ShapeDtypeStruct((_NSLOT, _D), jnp.float32),
    )(be, act, xs, w_gate, w_up, w_down)


def _combine_body(y_hbm, d1_hbm, d2_hbm, r1_hbm, r2_hbm, out_hbm,
                  idx_v, rw_v, yb1, yb2, sem):
    wid = lax.axis_index("s") * 2 + lax.axis_index("c")
    base = wid * _TPW
    for c in range(_TPW // _CH2):
        off = base + c * _CH2
        pltpu.sync_copy(d1_hbm.at[pl.ds(off, _CH2)], idx_v.at[0])
        pltpu.sync_copy(d2_hbm.at[pl.ds(off, _CH2)], idx_v.at[1])
        pltpu.sync_copy(r1_hbm.at[pl.ds(off, _CH2)], rw_v.at[0])
        pltpu.sync_copy(r2_hbm.at[pl.ds(off, _CH2)], rw_v.at[1])
        cp1 = pltpu.async_copy(y_hbm.at[idx_v.at[0]], yb1, sem)
        cp2 = pltpu.async_copy(y_hbm.at[idx_v.at[1]], yb2, sem)
        cp1.wait()
        cp2.wait()

        def row(i, carry):
            zero16 = jnp.zeros((16,), jnp.int32)
            one16 = jnp.ones((16,), jnp.int32)
            i16 = jnp.full((16,), i, jnp.int32)
            w1 = plsc.load_gather(rw_v, [zero16, i16])
            w2 = plsc.load_gather(rw_v, [one16, i16])

            def col(j, carry2):
                v = yb1[i, pl.ds(j * 16, 16)] * w1 \
                    + yb2[i, pl.ds(j * 16, 16)] * w2
                yb1[i, pl.ds(j * 16, 16)] = v
                return carry2

            lax.fori_loop(0, _D // 16, col, 0)
            return carry

        lax.fori_loop(0, _CH2, row, 0)
        pltpu.sync_copy(yb1, out_hbm.at[pl.ds(off, _CH2)])


_combine = functools.partial(
    pl.kernel,
    out_type=jax.ShapeDtypeStruct((_N, _D), jnp.float32),
    mesh=plsc.VectorSubcoreMesh(core_axis_name="c", subcore_axis_name="s"),
    scratch_types=[
        pltpu.VMEM((2, _CH2), jnp.int32),
        pltpu.VMEM((2, _CH2), jnp.float32),
        pltpu.VMEM((_CH2, _D), jnp.float32),
        pltpu.VMEM((_CH2, _D), jnp.float32),
        pltpu.SemaphoreType.DMA,
    ],
)(_combine_body)


def kernel(hidden_states, router_w, w_gate, w_up, w_down):
    b, s, d = hidden_states.shape
    x = hidden_states.reshape(-1, d)
    dest1, dest2, rw1, rw2, meta = _router(x, router_w)
    d1 = dest1.reshape(-1)
    d2 = dest2.reshape(-1)
    r1 = rw1.reshape(-1)
    r2 = rw2.reshape(-1)
    be = meta[0, :_NB]
    act = meta[1, :_NB]
    sorted_x = _dispatch(x, d1, d2)
    y = _ffn(be, act, sorted_x, w_gate, w_up, w_down)
    out = _combine(y, d1, d2, r1, r2)
    return out.reshape(b, s, d)


# speedup vs baseline: 10.1764x; 10.1764x over previous
import functools

import jax
import jax.numpy as jnp
from jax import lax
from jax.experimental import pallas as pl
from jax.experimental.pallas import tpu as pltpu
from jax.experimental.pallas import tpu_sc as plsc

_E = 16
_K = 2
_D = 1024
_FF = 768
_N = 4096
_BLK = 256
_NB = 48
_NSLOT = _NB * _BLK
_NBPAD = 64

_NW = 32
_TPW = _N // _NW
_CH1 = 64
_CH2 = 16


def _router_body(x_ref, w_ref, d1_ref, d2_ref, r1_ref, r2_ref, meta_ref):
    x = x_ref[...]
    w = w_ref[...]
    logits = lax.dot_general(x, w, (((1,), (1,)), ((), ())),
                             preferred_element_type=jnp.float32)
    lane = lax.broadcasted_iota(jnp.int32, (_N, _E), 1)
    m1 = jnp.max(logits, axis=1, keepdims=True)
    a1 = jnp.min(jnp.where(logits == m1, lane, _E), axis=1, keepdims=True)
    masked = jnp.where(lane == a1, jnp.finfo(jnp.float32).min, logits)
    m2 = jnp.max(masked, axis=1, keepdims=True)
    a2 = jnp.min(jnp.where(masked == m2, lane, _E), axis=1, keepdims=True)
    e2 = jnp.exp(m2 - m1)
    w1 = 1.0 / (1.0 + e2)
    w2 = e2 / (1.0 + e2)

    oh1 = lane == a1
    oh2 = lane == a2
    cnt = oh1.astype(jnp.int32) + oh2.astype(jnp.int32)
    cs = cnt
    k = 1
    while k < _N:
        cs = cs + jnp.concatenate(
            [jnp.zeros((k, _E), jnp.int32), cs[:-k, :]], axis=0)
        k *= 2
    excl = cs - cnt
    count = cs[_N - 1:_N, :]
    nblk = (count + (_BLK - 1)) // _BLK
    bs = nblk
    k = 1
    while k < _E:
        bs = bs + jnp.concatenate(
            [jnp.zeros((1, k), jnp.int32), bs[:, :-k]], axis=1)
        k *= 2
    bstart = bs - nblk
    pstart = bstart * _BLK

    d1 = jnp.sum(jnp.where(oh1, pstart + excl, 0), axis=1)
    d2 = jnp.sum(jnp.where(oh2, pstart + excl, 0), axis=1)
    d1_ref[...] = d1.reshape(_N // 128, 128)
    d2_ref[...] = d2.reshape(_N // 128, 128)
    r1_ref[...] = jnp.broadcast_to(w1, (_N, 128))
    r2_ref[...] = jnp.broadcast_to(w2, (_N, 128))

    bi = lax.broadcasted_iota(jnp.int32, (_NBPAD, _E), 0)
    lane_b = lax.broadcasted_iota(jnp.int32, (_NBPAD, _E), 1)
    act_mask = (bi >= bstart) & (bi < bstart + nblk)
    be = jnp.sum(jnp.where(act_mask, lane_b, 0), axis=1)
    active = jnp.sum(act_mask.astype(jnp.int32), axis=1)
    lane_r = lax.broadcasted_iota(jnp.int32, (1, _E), 1)
    lastexp = jnp.max(jnp.where(nblk > 0, lane_r, 0))
    be = jnp.where(active > 0, be, lastexp)
    meta_ref[...] = jnp.concatenate(
        [be.reshape(1, _NBPAD), active.reshape(1, _NBPAD)], axis=0)


def _router(x, router_w):
    return pl.pallas_call(
        _router_body,
        out_shape=[
            jax.ShapeDtypeStruct((_N // 128, 128), jnp.int32),
            jax.ShapeDtypeStruct((_N // 128, 128), jnp.int32),
            jax.ShapeDtypeStruct((_N, 128), jnp.float32),
            jax.ShapeDtypeStruct((_N, 128), jnp.float32),
            jax.ShapeDtypeStruct((2, _NBPAD), jnp.int32),
        ],
    )(x, router_w)


def _dispatch_body(x_hbm, d1_hbm, d2_hbm, r1_hbm, r2_hbm,
                   out_hbm, w_hbm, idx_v, xbuf, rbuf1, rbuf2, sem):
    wid = lax.axis_index("s") * 2 + lax.axis_index("c")
    base = wid * _TPW
    for c in range(_TPW // _CH1):
        off = base + c * _CH1
        pltpu.sync_copy(d1_hbm.at[pl.ds(off, _CH1)], idx_v.at[0])
        pltpu.sync_copy(d2_hbm.at[pl.ds(off, _CH1)], idx_v.at[1])
        pltpu.sync_copy(x_hbm.at[pl.ds(off, _CH1)], xbuf)
        pltpu.sync_copy(r1_hbm.at[pl.ds(off, _CH1)], rbuf1)
        pltpu.sync_copy(r2_hbm.at[pl.ds(off, _CH1)], rbuf2)
        cp1 = pltpu.async_copy(xbuf, out_hbm.at[idx_v.at[0]], sem)
        cp2 = pltpu.async_copy(xbuf, out_hbm.at[idx_v.at[1]], sem)
        cp3 = pltpu.async_copy(rbuf1, w_hbm.at[idx_v.at[0]], sem)
        cp4 = pltpu.async_copy(rbuf2, w_hbm.at[idx_v.at[1]], sem)
        cp1.wait()
        cp2.wait()
        cp3.wait()
        cp4.wait()


def _dispatch(x, d1, d2, r1, r2):
    f = pl.kernel(
        _dispatch_body,
        out_type=[
            jax.ShapeDtypeStruct((_NSLOT, _D), jnp.float32),
            jax.ShapeDtypeStruct((_NSLOT, 128), jnp.float32),
        ],
        mesh=plsc.VectorSubcoreMesh(core_axis_name="c", subcore_axis_name="s"),
        scratch_types=[
            pltpu.VMEM((2, _CH1), jnp.int32),
            pltpu.VMEM((_CH1, _D), jnp.float32),
            pltpu.VMEM((_CH1, 128), jnp.float32),
            pltpu.VMEM((_CH1, 128), jnp.float32),
            pltpu.SemaphoreType.DMA,
        ],
    )
    return f(x, d1, d2, r1, r2)


def _ffn_body(be_ref, act_ref, xs_ref, ws_ref, wg_ref, wu_ref, wd_ref, y_ref):
    i = pl.program_id(0)

    @pl.when(act_ref[i] > 0)
    def _():
        xb = xs_ref[...]
        g = lax.dot_general(xb, wg_ref[0], (((1,), (1,)), ((), ())),
                            preferred_element_type=jnp.float32)
        u = lax.dot_general(xb, wu_ref[0], (((1,), (1,)), ((), ())),
                            preferred_element_type=jnp.float32)
        h = (g * jax.nn.sigmoid(g)) * u
        y = lax.dot_general(h, wd_ref[0], (((1,), (1,)), ((), ())),
                            preferred_element_type=jnp.float32)
        y_ref[...] = y * ws_ref[...][:, 0:1]


def _ffn(be, act, xs, wslot, w_gate, w_up, w_down):
    grid_spec = pltpu.PrefetchScalarGridSpec(
        num_scalar_prefetch=2,
        grid=(_NB,),
        in_specs=[
            pl.BlockSpec((_BLK, _D), lambda i, be, act: (i, 0)),
            pl.BlockSpec((_BLK, 128), lambda i, be, act: (i, 0)),
            pl.BlockSpec((1, _FF, _D), lambda i, be, act: (be[i], 0, 0)),
            pl.BlockSpec((1, _FF, _D), lambda i, be, act: (be[i], 0, 0)),
            pl.BlockSpec((1, _D, _FF), lambda i, be, act: (be[i], 0, 0)),
        ],
        out_specs=pl.BlockSpec((_BLK, _D), lambda i, be, act: (i, 0)),
    )
    return pl.pallas_call(
        _ffn_body,
        grid_spec=grid_spec,
        out_shape=jax.ShapeDtypeStruct((_NSLOT, _D), jnp.float32),
    )(be, act, xs, wslot, w_gate, w_up, w_down)


def _combine_body(y_hbm, d1_hbm, d2_hbm, out_hbm, idx_v, yb1, yb2, sem):
    wid = lax.axis_index("s") * 2 + lax.axis_index("c")
    base = wid * _TPW
    for c in range(_TPW // _CH2):
        off = base + c * _CH2
        pltpu.sync_copy(d1_hbm.at[pl.ds(off, _CH2)], idx_v.at[0])
        pltpu.sync_copy(d2_hbm.at[pl.ds(off, _CH2)], idx_v.at[1])
        cp1 = pltpu.async_copy(y_hbm.at[idx_v.at[0]], yb1, sem)
        cp2 = pltpu.async_copy(y_hbm.at[idx_v.at[1]], yb2, sem)
        cp1.wait()
        cp2.wait()

        def row(i, carry):
            def col(j, carry2):
                v = yb1[i, pl.ds(j * 16, 16)] + yb2[i, pl.ds(j * 16, 16)]
                yb1[i, pl.ds(j * 16, 16)] = v
                return carry2

            lax.fori_loop(0, _D // 16, col, 0)
            return carry

        lax.fori_loop(0, _CH2, row, 0)
        pltpu.sync_copy(yb1, out_hbm.at[pl.ds(off, _CH2)])


def _combine(y, d1, d2):
    f = pl.kernel(
        _combine_body,
        out_type=jax.ShapeDtypeStruct((_N, _D), jnp.float32),
        mesh=plsc.VectorSubcoreMesh(core_axis_name="c", subcore_axis_name="s"),
        scratch_types=[
            pltpu.VMEM((2, _CH2), jnp.int32),
            pltpu.VMEM((_CH2, _D), jnp.float32),
            pltpu.VMEM((_CH2, _D), jnp.float32),
            pltpu.SemaphoreType.DMA,
        ],
    )
    return f(y, d1, d2)


def kernel(hidden_states, router_w, w_gate, w_up, w_down):
    b, s, d = hidden_states.shape
    x = hidden_states.reshape(-1, d)
    dest1, dest2, r1, r2, meta = _router(x, router_w)
    d1 = dest1.reshape(-1)
    d2 = dest2.reshape(-1)
    be = meta[0, :_NB]
    act = meta[1, :_NB]
    sorted_x, wslot = _dispatch(x, d1, d2, r1, r2)
    y = _ffn(be, act, sorted_x, wslot, w_gate, w_up, w_down)
    out = _combine(y, d1, d2)
    return out.reshape(b, s, d)


# speedup vs baseline: 11.1548x; 1.0962x over previous
import functools

import jax
import jax.numpy as jnp
from jax import lax
from jax.experimental import pallas as pl
from jax.experimental.pallas import tpu as pltpu
from jax.experimental.pallas import tpu_sc as plsc

_E = 16
_K = 2
_D = 1024
_FF = 768
_N = 4096
_BLK = 256
_NB = 48
_NSLOT = _NB * _BLK
_NBPAD = 64

_NW = 32
_TPW = _N // _NW
_CH1 = 64
_CH2 = 32


def _router_body(x_ref, w_ref, d1_ref, d2_ref, r1_ref, r2_ref, meta_ref):
    x = x_ref[...]
    w = w_ref[...]
    logits = lax.dot_general(x, w, (((1,), (1,)), ((), ())),
                             preferred_element_type=jnp.float32)
    lane = lax.broadcasted_iota(jnp.int32, (_N, _E), 1)
    m1 = jnp.max(logits, axis=1, keepdims=True)
    a1 = jnp.min(jnp.where(logits == m1, lane, _E), axis=1, keepdims=True)
    masked = jnp.where(lane == a1, jnp.finfo(jnp.float32).min, logits)
    m2 = jnp.max(masked, axis=1, keepdims=True)
    a2 = jnp.min(jnp.where(masked == m2, lane, _E), axis=1, keepdims=True)
    e2 = jnp.exp(m2 - m1)
    w1 = 1.0 / (1.0 + e2)
    w2 = e2 / (1.0 + e2)

    oh1 = lane == a1
    oh2 = lane == a2
    cnt = oh1.astype(jnp.int32) + oh2.astype(jnp.int32)
    cs = cnt
    k = 1
    while k < _N:
        cs = cs + jnp.concatenate(
            [jnp.zeros((k, _E), jnp.int32), cs[:-k, :]], axis=0)
        k *= 2
    excl = cs - cnt
    count = cs[_N - 1:_N, :]
    nblk = (count + (_BLK - 1)) // _BLK
    bs = nblk
    k = 1
    while k < _E:
        bs = bs + jnp.concatenate(
            [jnp.zeros((1, k), jnp.int32), bs[:, :-k]], axis=1)
        k *= 2
    bstart = bs - nblk
    pstart = bstart * _BLK

    d1 = jnp.sum(jnp.where(oh1, pstart + excl, 0), axis=1)
    d2 = jnp.sum(jnp.where(oh2, pstart + excl, 0), axis=1)
    d1_ref[...] = d1.reshape(_N // 128, 128)
    d2_ref[...] = d2.reshape(_N // 128, 128)
    r1_ref[...] = jnp.broadcast_to(w1, (_N, 128))
    r2_ref[...] = jnp.broadcast_to(w2, (_N, 128))

    bi = lax.broadcasted_iota(jnp.int32, (_NBPAD, _E), 0)
    lane_b = lax.broadcasted_iota(jnp.int32, (_NBPAD, _E), 1)
    act_mask = (bi >= bstart) & (bi < bstart + nblk)
    be = jnp.sum(jnp.where(act_mask, lane_b, 0), axis=1)
    active = jnp.sum(act_mask.astype(jnp.int32), axis=1)
    lane_r = lax.broadcasted_iota(jnp.int32, (1, _E), 1)
    lastexp = jnp.max(jnp.where(nblk > 0, lane_r, 0))
    be = jnp.where(active > 0, be, lastexp)
    meta_ref[...] = jnp.concatenate(
        [be.reshape(1, _NBPAD), active.reshape(1, _NBPAD)], axis=0)


def _router(x, router_w):
    return pl.pallas_call(
        _router_body,
        out_shape=[
            jax.ShapeDtypeStruct((_N // 128, 128), jnp.int32),
            jax.ShapeDtypeStruct((_N // 128, 128), jnp.int32),
            jax.ShapeDtypeStruct((_N, 128), jnp.float32),
            jax.ShapeDtypeStruct((_N, 128), jnp.float32),
            jax.ShapeDtypeStruct((2, _NBPAD), jnp.int32),
        ],
    )(x, router_w)


def _dispatch_body(x_hbm, d1_hbm, d2_hbm, r1_hbm, r2_hbm,
                   out_hbm, w_hbm, idx_v, xbuf, rbuf1, rbuf2, sem):
    wid = lax.axis_index("s") * 2 + lax.axis_index("c")
    base = wid * _TPW
    for c in range(_TPW // _CH1):
        off = base + c * _CH1
        pltpu.sync_copy(d1_hbm.at[pl.ds(off, _CH1)], idx_v.at[0])
        pltpu.sync_copy(d2_hbm.at[pl.ds(off, _CH1)], idx_v.at[1])
        pltpu.sync_copy(x_hbm.at[pl.ds(off, _CH1)], xbuf)
        pltpu.sync_copy(r1_hbm.at[pl.ds(off, _CH1)], rbuf1)
        pltpu.sync_copy(r2_hbm.at[pl.ds(off, _CH1)], rbuf2)
        cp1 = pltpu.async_copy(xbuf, out_hbm.at[idx_v.at[0]], sem)
        cp2 = pltpu.async_copy(xbuf, out_hbm.at[idx_v.at[1]], sem)
        cp3 = pltpu.async_copy(rbuf1, w_hbm.at[idx_v.at[0]], sem)
        cp4 = pltpu.async_copy(rbuf2, w_hbm.at[idx_v.at[1]], sem)
        cp1.wait()
        cp2.wait()
        cp3.wait()
        cp4.wait()


def _dispatch(x, d1, d2, r1, r2):
    f = pl.kernel(
        _dispatch_body,
        out_type=[
            jax.ShapeDtypeStruct((_NSLOT, _D), jnp.float32),
            jax.ShapeDtypeStruct((_NSLOT, 128), jnp.float32),
        ],
        mesh=plsc.VectorSubcoreMesh(core_axis_name="c", subcore_axis_name="s"),
        scratch_types=[
            pltpu.VMEM((2, _CH1), jnp.int32),
            pltpu.VMEM((_CH1, _D), jnp.float32),
            pltpu.VMEM((_CH1, 128), jnp.float32),
            pltpu.VMEM((_CH1, 128), jnp.float32),
            pltpu.SemaphoreType.DMA,
        ],
    )
    return f(x, d1, d2, r1, r2)


def _ffn_body(be_ref, act_ref, xs_ref, ws_ref, wg_ref, wu_ref, wd_ref, y_ref):
    i = pl.program_id(0)

    @pl.when(act_ref[i] > 0)
    def _():
        xb = xs_ref[...]
        g = lax.dot_general(xb, wg_ref[0], (((1,), (1,)), ((), ())),
                            preferred_element_type=jnp.float32)
        u = lax.dot_general(xb, wu_ref[0], (((1,), (1,)), ((), ())),
                            preferred_element_type=jnp.float32)
        h = (g * jax.nn.sigmoid(g)) * u
        y = lax.dot_general(h, wd_ref[0], (((1,), (1,)), ((), ())),
                            preferred_element_type=jnp.float32)
        y_ref[...] = y * ws_ref[...][:, 0:1]


def _ffn(be, act, xs, wslot, w_gate, w_up, w_down):
    grid_spec = pltpu.PrefetchScalarGridSpec(
        num_scalar_prefetch=2,
        grid=(_NB,),
        in_specs=[
            pl.BlockSpec((_BLK, _D), lambda i, be, act: (i, 0)),
            pl.BlockSpec((_BLK, 128), lambda i, be, act: (i, 0)),
            pl.BlockSpec((1, _FF, _D), lambda i, be, act: (be[i], 0, 0)),
            pl.BlockSpec((1, _FF, _D), lambda i, be, act: (be[i], 0, 0)),
            pl.BlockSpec((1, _D, _FF), lambda i, be, act: (be[i], 0, 0)),
        ],
        out_specs=pl.BlockSpec((_BLK, _D), lambda i, be, act: (i, 0)),
    )
    return pl.pallas_call(
        _ffn_body,
        grid_spec=grid_spec,
        out_shape=jax.ShapeDtypeStruct((_NSLOT, _D), jnp.float32),
    )(be, act, xs, wslot, w_gate, w_up, w_down)


def _combine_body(y_hbm, d1_hbm, d2_hbm, ya_hbm, yb_hbm, idx_v, yb1, yb2, sem):
    wid = lax.axis_index("s") * 2 + lax.axis_index("c")
    base = wid * _TPW
    for c in range(_TPW // _CH2):
        off = base + c * _CH2
        pltpu.sync_copy(d1_hbm.at[pl.ds(off, _CH2)], idx_v.at[0])
        pltpu.sync_copy(d2_hbm.at[pl.ds(off, _CH2)], idx_v.at[1])
        cp1 = pltpu.async_copy(y_hbm.at[idx_v.at[0]], yb1, sem)
        cp2 = pltpu.async_copy(y_hbm.at[idx_v.at[1]], yb2, sem)
        cp1.wait()
        cp2.wait()
        cp3 = pltpu.async_copy(yb1, ya_hbm.at[pl.ds(off, _CH2)], sem)
        cp4 = pltpu.async_copy(yb2, yb_hbm.at[pl.ds(off, _CH2)], sem)
        cp3.wait()
        cp4.wait()


def _combine(y, d1, d2):
    f = pl.kernel(
        _combine_body,
        out_type=[
            jax.ShapeDtypeStruct((_N, _D), jnp.float32),
            jax.ShapeDtypeStruct((_N, _D), jnp.float32),
        ],
        mesh=plsc.VectorSubcoreMesh(core_axis_name="c", subcore_axis_name="s"),
        scratch_types=[
            pltpu.VMEM((2, _CH2), jnp.int32),
            pltpu.VMEM((_CH2, _D), jnp.float32),
            pltpu.VMEM((_CH2, _D), jnp.float32),
            pltpu.SemaphoreType.DMA,
        ],
    )
    return f(y, d1, d2)


def _add_body(a_ref, b_ref, o_ref):
    o_ref[...] = a_ref[...] + b_ref[...]


def _add(a, b):
    blk = _N // 8
    return pl.pallas_call(
        _add_body,
        grid=(8,),
        in_specs=[
            pl.BlockSpec((blk, _D), lambda i: (i, 0)),
            pl.BlockSpec((blk, _D), lambda i: (i, 0)),
        ],
        out_specs=pl.BlockSpec((blk, _D), lambda i: (i, 0)),
        out_shape=jax.ShapeDtypeStruct((_N, _D), jnp.float32),
    )(a, b)


def kernel(hidden_states, router_w, w_gate, w_up, w_down):
    b, s, d = hidden_states.shape
    x = hidden_states.reshape(-1, d)
    dest1, dest2, r1, r2, meta = _router(x, router_w)
    d1 = dest1.reshape(-1)
    d2 = dest2.reshape(-1)
    be = meta[0, :_NB]
    act = meta[1, :_NB]
    sorted_x, wslot = _dispatch(x, d1, d2, r1, r2)
    y = _ffn(be, act, sorted_x, wslot, w_gate, w_up, w_down)
    ya, yb = _combine(y, d1, d2)
    out = _add(ya, yb)
    return out.reshape(b, s, d)
